# baseline (speedup 1.0000x reference)
import functools

import jax
import jax.numpy as jnp
from jax import lax
from jax.experimental import pallas as pl
from jax.experimental.pallas import tpu as pltpu
from jax.experimental.pallas import tpu_sc as plsc

NC = 2
NS = 16
NW = NC * NS
CH = 128
NBUF = 16


def _gather_call(x2d, emb_table, n_idx, hidden):
    chunks_per_w = n_idx // (NW * CH)
    rows_per_w = n_idx // NW
    ngroups = chunks_per_w // NBUF
    mesh = plsc.VectorSubcoreMesh(core_axis_name="c", subcore_axis_name="s")

    @functools.partial(
        pl.kernel,
        mesh=mesh,
        out_type=jax.ShapeDtypeStruct((n_idx, hidden), jnp.float32),
        compiler_params=pltpu.CompilerParams(use_tc_tiling_on_sc=False),
        scratch_types=[
            pltpu.VMEM((chunks_per_w, CH), jnp.int32),
            pltpu.VMEM((NBUF, CH, hidden), jnp.float32),
            pltpu.SemaphoreType.DMA,
            pltpu.SemaphoreType.DMA,
        ],
    )
    def gather_k(x_hbm, table_hbm, out_hbm, idx_v, rows_v, gsem, osem):
        wid = lax.axis_index("s") * NC + lax.axis_index("c")
        base = wid * rows_per_w
        pltpu.sync_copy(x_hbm.at[pl.ds(wid * chunks_per_w, chunks_per_w)], idx_v)
        for b in range(NBUF):
            pltpu.async_copy(table_hbm.at[idx_v.at[b]], rows_v.at[b], gsem)

        def grp(g, carry):
            j0 = g * NBUF
            for b in range(NBUF):
                pltpu.make_async_copy(
                    table_hbm.at[idx_v.at[j0 + b]], rows_v.at[b], gsem
                ).wait()
                pltpu.async_copy(
                    rows_v.at[b], out_hbm.at[pl.ds(base + (j0 + b) * CH, CH)], osem
                )
            for b in range(NBUF):
                pltpu.make_async_copy(
                    rows_v.at[b], out_hbm.at[pl.ds(base + (j0 + b) * CH, CH)], osem
                ).wait()

                @pl.when(g + 1 < ngroups)
                def _():
                    pltpu.async_copy(
                        table_hbm.at[idx_v.at[j0 + NBUF + b]], rows_v.at[b], gsem
                    )

            return carry

        lax.fori_loop(0, ngroups, grp, 0)

    return gather_k(x2d, emb_table)


def _dense_body(nsub, e_ref, w_ref, b_ref, o_ref):
    rows_pb = e_ref.shape[0]
    npair = rows_pb // nsub
    ncols = w_ref.shape[1]
    ncls2 = ncols // nsub
    e3 = e_ref[...].reshape(npair, nsub, 128)
    w3 = w_ref[...].reshape(128, nsub, ncls2)
    acc = jnp.dot(e3[:, 0, :], w3[:, 0, :], preferred_element_type=jnp.float32)
    for j in range(1, nsub):
        acc += jnp.dot(e3[:, j, :], w3[:, j, :], preferred_element_type=jnp.float32)
    logits = acc + b_ref[...]
    ncls = ncls2 // 2
    out = []
    for h in range(2):
        lg = logits[:, h * ncls:(h + 1) * ncls]
        m = jnp.max(lg, axis=-1, keepdims=True)
        ez = jnp.exp(lg - m)
        lse = jnp.log(jnp.sum(ez, axis=-1, keepdims=True)) + m
        out.append(lg - lse)
    o_ref[...] = jnp.concatenate(out, axis=1)


def _dense_call(e128, w30, b2d, batch, nsub, ncls):
    npair_tot = batch // 2
    npair_blk = 256
    rows_pb = npair_blk * nsub
    grid = npair_tot // npair_blk
    ncols = nsub * 2 * ncls
    body = functools.partial(_dense_body, nsub)
    return pl.pallas_call(
        body,
        grid=(grid,),
        in_specs=[
            pl.BlockSpec((rows_pb, 128), lambda i: (i, 0)),
            pl.BlockSpec((128, ncols), lambda i: (0, 0)),
            pl.BlockSpec((1, 2 * ncls), lambda i: (0, 0)),
        ],
        out_specs=pl.BlockSpec((npair_blk, 2 * ncls), lambda i: (i, 0)),
        out_shape=jax.ShapeDtypeStruct((npair_tot, 2 * ncls), jnp.float32),
    )(e128, w30, b2d)


def kernel(x, emb_table, fc1_w, fc1_b):
    batch, hist = x.shape
    _, hidden = emb_table.shape
    ncls, hidden2 = fc1_w.shape
    n_idx = batch * hist
    nsub = 2 * hidden2 // 128
    x2d = x.reshape(n_idx // CH, CH).astype(jnp.int32)
    embeds = _gather_call(x2d, emb_table, n_idx, hidden)
    e128 = embeds.reshape(n_idx * hidden // 128, 128)
    wt = fc1_w.T
    z = jnp.zeros_like(wt)
    w2 = jnp.concatenate(
        [jnp.concatenate([wt, z], axis=1), jnp.concatenate([z, wt], axis=1)], axis=0
    )
    w30 = w2.reshape(nsub, 128, 2 * ncls).transpose(1, 0, 2).reshape(128, nsub * 2 * ncls)
    b2d = jnp.tile(fc1_b, 2).reshape(1, 2 * ncls)
    out6 = _dense_call(e128, w30, b2d, batch, nsub, ncls)
    return out6.reshape(batch, ncls)


# speedup vs baseline: 1.0102x; 1.0102x over previous
import functools

import jax
import jax.numpy as jnp
from jax import lax
from jax.experimental import pallas as pl
from jax.experimental.pallas import tpu as pltpu
from jax.experimental.pallas import tpu_sc as plsc

NC = 2
NS = 16
NW = NC * NS
CH = 128
NBUF = 8


def _gather_call(x2d, emb_table, n_idx, hidden):
    chunks_per_w = n_idx // (NW * CH)
    rows_per_w = n_idx // NW
    ngroups = chunks_per_w // NBUF
    mesh = plsc.VectorSubcoreMesh(core_axis_name="c", subcore_axis_name="s")

    @functools.partial(
        pl.kernel,
        mesh=mesh,
        out_type=jax.ShapeDtypeStruct((n_idx, hidden), jnp.float32),
        compiler_params=pltpu.CompilerParams(use_tc_tiling_on_sc=False),
        scratch_types=[
            pltpu.VMEM((chunks_per_w, CH), jnp.int32),
            pltpu.VMEM((NBUF, CH, hidden), jnp.float32),
            pltpu.SemaphoreType.DMA,
            pltpu.SemaphoreType.DMA,
        ],
    )
    def gather_k(x_hbm, table_hbm, out_hbm, idx_v, rows_v, gsem, osem):
        wid = lax.axis_index("s") * NC + lax.axis_index("c")
        base = wid * rows_per_w
        pltpu.sync_copy(x_hbm.at[pl.ds(wid * chunks_per_w, chunks_per_w)], idx_v)
        for b in range(NBUF):
            pltpu.async_copy(table_hbm.at[idx_v.at[b]], rows_v.at[b], gsem)

        def grp(g, carry):
            j0 = g * NBUF
            for b in range(NBUF):
                pltpu.make_async_copy(
                    table_hbm.at[idx_v.at[j0 + b]], rows_v.at[b], gsem
                ).wait()
                pltpu.async_copy(
                    rows_v.at[b], out_hbm.at[pl.ds(base + (j0 + b) * CH, CH)], osem
                )
            for b in range(NBUF):
                pltpu.make_async_copy(
                    rows_v.at[b], out_hbm.at[pl.ds(base + (j0 + b) * CH, CH)], osem
                ).wait()

                @pl.when(g + 1 < ngroups)
                def _():
                    pltpu.async_copy(
                        table_hbm.at[idx_v.at[j0 + NBUF + b]], rows_v.at[b], gsem
                    )

            return carry

        lax.fori_loop(0, ngroups, grp, 0)

    return gather_k(x2d, emb_table)


def _dense_body(nsub, e_ref, w_ref, b_ref, o_ref):
    rows_pb = e_ref.shape[0]
    npair = rows_pb // nsub
    ncols = w_ref.shape[1]
    ncls2 = ncols // nsub
    e3 = e_ref[...].reshape(npair, nsub, 128)
    w3 = w_ref[...].reshape(128, nsub, ncls2)
    acc = jnp.dot(e3[:, 0, :], w3[:, 0, :], preferred_element_type=jnp.float32)
    for j in range(1, nsub):
        acc += jnp.dot(e3[:, j, :], w3[:, j, :], preferred_element_type=jnp.float32)
    logits = acc + b_ref[...]
    ncls = ncls2 // 2
    out = []
    for h in range(2):
        lg = logits[:, h * ncls:(h + 1) * ncls]
        m = jnp.max(lg, axis=-1, keepdims=True)
        ez = jnp.exp(lg - m)
        lse = jnp.log(jnp.sum(ez, axis=-1, keepdims=True)) + m
        out.append(lg - lse)
    o_ref[...] = jnp.concatenate(out, axis=1)


def _dense_call(e128, w30, b2d, batch, nsub, ncls):
    npair_tot = batch // 2
    npair_blk = 512
    rows_pb = npair_blk * nsub
    grid = npair_tot // npair_blk
    ncols = nsub * 2 * ncls
    body = functools.partial(_dense_body, nsub)
    return pl.pallas_call(
        body,
        grid=(grid,),
        in_specs=[
            pl.BlockSpec((rows_pb, 128), lambda i: (i, 0)),
            pl.BlockSpec((128, ncols), lambda i: (0, 0)),
            pl.BlockSpec((1, 2 * ncls), lambda i: (0, 0)),
        ],
        out_specs=pl.BlockSpec((npair_blk, 2 * ncls), lambda i: (i, 0)),
        out_shape=jax.ShapeDtypeStruct((npair_tot, 2 * ncls), jnp.float32),
    )(e128, w30, b2d)


def kernel(x, emb_table, fc1_w, fc1_b):
    batch, hist = x.shape
    _, hidden = emb_table.shape
    ncls, hidden2 = fc1_w.shape
    n_idx = batch * hist
    nsub = 2 * hidden2 // 128
    x2d = x.reshape(n_idx // CH, CH).astype(jnp.int32)
    embeds = _gather_call(x2d, emb_table, n_idx, hidden)
    e128 = embeds.reshape(n_idx * hidden // 128, 128)
    wt = fc1_w.T
    z = jnp.zeros_like(wt)
    w2 = jnp.concatenate(
        [jnp.concatenate([wt, z], axis=1), jnp.concatenate([z, wt], axis=1)], axis=0
    )
    w30 = w2.reshape(nsub, 128, 2 * ncls).transpose(1, 0, 2).reshape(128, nsub * 2 * ncls)
    b2d = jnp.tile(fc1_b, 2).reshape(1, 2 * ncls)
    out6 = _dense_call(e128, w30, b2d, batch, nsub, ncls)
    return out6.reshape(batch, ncls)
